# baseline (speedup 1.0000x reference)
import functools

import jax
import jax.numpy as jnp
import numpy as np
from jax import lax
from jax.experimental import pallas as pl
from jax.experimental.pallas import tpu as pltpu
from jax.experimental.pallas import tpu_sc as plsc

NSP = 1024
NTILES = 32
CHUNK = 12500
QPB = 4
K = 5000
W = 256
TPAD = NSP + W


_I16 = lambda: lax.broadcasted_iota(jnp.int32, (16,), 0)


def _stat_combine(a, b, min_m, max_m):
    return jnp.where(min_m, jnp.minimum(a, b),
                     jnp.where(max_m, jnp.maximum(a, b), a + b))


def _sc_stats_body(xyz_hbm, idx_hbm, xyzn_hbm, cnt_hbm, xyz_v, idx_v, tbl_v,
                   sib_v, ctrx_v, ctry_v, ctrz_v, inv_v, cnt_v, shared_v):
    c = lax.axis_index("c")
    s = lax.axis_index("s")
    wid = c * 16 + s
    b = wid // QPB
    iota16 = _I16()
    min_m = iota16 < 4
    max_m = (iota16 >= 4) & (iota16 < 8)
    one_m = iota16 == 11
    zero_m = iota16 >= 12
    init_row = jnp.where(min_m, jnp.inf, jnp.where(max_m, -jnp.inf, 0.0))
    colp = jnp.where((iota16 < 12) & (iota16 % 4 != 3), iota16 % 4, 0)

    pltpu.sync_copy(xyz_hbm.at[wid], xyz_v)
    pltpu.sync_copy(idx_hbm.at[wid], idx_v)

    def _init(i, carry):
        tbl_v[pl.ds(i * 16, 16)] = init_row
        return carry
    lax.fori_loop(0, NSP, _init, 0)

    def _pt_one(p, sg):
        g = plsc.load_gather(xyz_v, [4 * p + colp])
        pv = jnp.where(one_m, 1.0, jnp.where(zero_m, 0.0, g))
        r = pl.ds(sg * 16, 16)
        tbl_v[r] = _stat_combine(tbl_v[r], pv, min_m, max_m)

    def _pt(k, carry):
        p0 = k * 16
        iv = idx_v[pl.ds(p0, 16)]
        for j in range(16):
            _pt_one(p0 + j, iv[j])
        return carry
    lax.fori_loop(0, CHUNK // 16, _pt, 0)
    ivt = idx_v[pl.ds(CHUNK - 16, 16)]
    for j in range(16 - CHUNK % 16, 16):
        _pt_one(CHUNK - 16 + j, ivt[j])

    pltpu.sync_copy(tbl_v, shared_v.at[s])
    plsc.subcore_barrier()
    base = (s // QPB) * QPB
    pltpu.sync_copy(shared_v.at[base], tbl_v)
    for o in range(1, QPB):
        pltpu.sync_copy(shared_v.at[base + o], sib_v)

        def _mrg(i, carry):
            r = pl.ds(i * 16, 16)
            tbl_v[r] = _stat_combine(tbl_v[r], sib_v[r], min_m, max_m)
            return carry
        lax.fori_loop(0, NSP, _mrg, 0)

    def _nt(k, carry):
        s16 = k * 16 + iota16

        def col(cix):
            return plsc.load_gather(tbl_v, [s16 * 16 + cix])
        diam = jnp.maximum(jnp.maximum(col(4) - col(0), col(5) - col(1)),
                           col(6) - col(2))
        ct = col(11)
        cc = jnp.maximum(ct, 1.0)
        sl = pl.ds(k * 16, 16)
        ctrx_v[sl] = col(8) / cc
        ctry_v[sl] = col(9) / cc
        ctrz_v[sl] = col(10) / cc
        inv_v[sl] = 1.0 / (diam + 0.01)
        cnt_v[sl] = ct
        return carry
    lax.fori_loop(0, NSP // 16, _nt, 0)

    def _xn(k, carry):
        p16 = jnp.minimum(k * 16 + iota16, CHUNK - 1)
        i16 = plsc.load_gather(idx_v, [p16])
        cx = plsc.load_gather(ctrx_v, [i16])
        cy = plsc.load_gather(ctry_v, [i16])
        cz = plsc.load_gather(ctrz_v, [i16])
        iv = plsc.load_gather(inv_v, [i16])
        xs = plsc.load_gather(xyz_v, [4 * p16])
        ys = plsc.load_gather(xyz_v, [4 * p16 + 1])
        zs = plsc.load_gather(xyz_v, [4 * p16 + 2])
        plsc.store_scatter(xyz_v, [4 * p16], (xs - cx) * iv)
        plsc.store_scatter(xyz_v, [4 * p16 + 1], (ys - cy) * iv)
        plsc.store_scatter(xyz_v, [4 * p16 + 2], (zs - cz) * iv)
        return carry
    lax.fori_loop(0, (CHUNK + 15) // 16, _xn, 0)

    pltpu.sync_copy(xyz_v, xyzn_hbm.at[wid])

    @pl.when(wid % QPB == 0)
    def _():
        pltpu.sync_copy(cnt_v, cnt_hbm.at[b])


SCCH = 128
NCH = CHUNK // SCCH
TAILP = CHUNK - NCH * SCCH


def _sc_segmax_body(proj_hbm, idx_hbm, slab_hbm, buf_v, idx_v, slab_v, acc_v):
    c = lax.axis_index("c")
    s = lax.axis_index("s")
    wid = c * 16 + s
    iota16 = _I16()
    ninf = jnp.where(iota16 >= 0, -jnp.inf, 0.0)

    pltpu.sync_copy(idx_hbm.at[wid], idx_v)

    def _clr(i, carry):
        slab_v[pl.ds(i * 16, 16)] = ninf
        return carry
    lax.fori_loop(0, NSP * 6, _clr, 0)
    for kk in range(6):
        acc_v[pl.ds(kk * 16, 16)] = ninf

    def _pt_one(off, sg, prev):
        @pl.when(sg != prev)
        def _flush():
            for kk in range(6):
                r = pl.ds(kk * 16, 16)
                slab_v[pl.ds(prev * 96 + kk * 16, 16)] = acc_v[r]
                acc_v[r] = ninf
        for kk in range(6):
            r = pl.ds(kk * 16, 16)
            acc_v[r] = jnp.maximum(acc_v[r], buf_v[pl.ds(off + kk * 16, 16)])
        return sg

    def _grp(q, g, prev):
        iv = idx_v[pl.ds(q * SCCH + g * 16, 16)]
        for j in range(16):
            prev = _pt_one((g * 16 + j) * 128, iv[j], prev)
        return prev

    def _chunk(q, prev):
        pltpu.sync_copy(proj_hbm.at[wid, pl.ds(q * SCCH * 128, SCCH * 128)],
                        buf_v)
        return lax.fori_loop(0, SCCH // 16, functools.partial(_grp, q), prev)

    iv0 = idx_v[pl.ds(0, 16)]
    prev = lax.fori_loop(0, NCH, _chunk, iv0[0])
    pltpu.sync_copy(proj_hbm.at[wid, pl.ds(NCH * SCCH * 128, TAILP * 128)],
                    buf_v.at[pl.ds(0, TAILP * 128)])
    for g in range(TAILP // 16):
        ivt = idx_v[pl.ds(NCH * SCCH + g * 16, 16)]
        for j in range(16):
            prev = _pt_one((g * 16 + j) * 128, ivt[j], prev)
    ivt = idx_v[pl.ds(CHUNK - 16, 16)]
    for j in range(16 - TAILP % 16, 16):
        prev = _pt_one((TAILP - 16 + j) * 128, ivt[j], prev)
    for kk in range(6):
        slab_v[pl.ds(prev * 96 + kk * 16, 16)] = acc_v[pl.ds(kk * 16, 16)]

    pltpu.sync_copy(slab_v, slab_hbm.at[wid])


def _lnk(x, g, b, eps=1e-5):
    m = jnp.mean(x, axis=-1, keepdims=True)
    v = jnp.mean((x - m) ** 2, axis=-1, keepdims=True)
    return (x - m) / jnp.sqrt(v + eps) * g + b


def _mlp1_body(xyz_ref, w1_ref, b1_ref, g1_ref, be1_ref, w2_ref, b2_ref,
               g2_ref, be2_ref, out_ref):
    h = xyz_ref[...] @ w1_ref[...] + b1_ref[...]
    h = jnp.maximum(_lnk(h, g1_ref[...], be1_ref[...]), 0.0)
    p = h @ w2_ref[...] + b2_ref[...]
    p = _lnk(p, g2_ref[...], be2_ref[...])
    out_ref[...] = jnp.concatenate(
        [p, jnp.zeros((p.shape[0], 32), jnp.float32)], axis=1)


def _pass_c_body(s0_ref, s1_ref, idx_ref, proj_ref, feat_ref, slab_ref,
                 cnt_ref, w3a_ref, w3b_ref, b3_ref, g3_ref, be3_ref, w4_ref,
                 w_out_ref, sp_out_ref, acc_ref, t_ref, g_ref, fw_ref,
                 seg_m_ref, *, nblk):
    i = pl.program_id(0)
    blk = i % nblk

    @pl.when(blk == 0)
    def _init():
        sl = slab_ref[0]
        seg = jnp.max(sl, axis=0)
        seg_m_ref[...] = seg
        seg_clean = jnp.where(jnp.isfinite(seg), seg, 0.0)
        t_ref[:NSP, :] = seg_clean @ w3b_ref[...] + b3_ref[...]
        t_ref[NSP:, :] = jnp.zeros((TPAD - NSP, t_ref.shape[1]), jnp.float32)
        acc_ref[...] = jnp.zeros_like(acc_ref)

    s0 = s0_ref[0, 0, 0] & ~7
    s1 = s1_ref[0, 0, 0]

    for j in range(4):
        base = s0 + j * W

        @pl.when(base <= s1)
        def _gather():
            idx_col = idx_ref[0, 0, :][:, None]
            iota_w = base + lax.broadcasted_iota(jnp.int32, (1, W), 1)
            oh = (idx_col == iota_w).astype(jnp.float32)
            contrib = jnp.dot(oh, t_ref[pl.ds(base, W), :],
                              preferred_element_type=jnp.float32)
            if j == 0:
                g_ref[...] = contrib
            else:
                g_ref[...] += contrib

    y = proj_ref[...] @ w3a_ref[...] + g_ref[...]
    h2 = jnp.maximum(_lnk(y, g3_ref[...], be3_ref[...]), 0.0)
    w = jax.nn.sigmoid(h2 @ w4_ref[...]) * 2.0
    w_out_ref[...] = w
    fw_ref[...] = feat_ref[...] * w

    for j in range(4):
        base = s0 + j * W

        @pl.when(base <= s1)
        def _scatter():
            idx_row = idx_ref[0, 0, :][None, :]
            iota_c = base + lax.broadcasted_iota(jnp.int32, (W, 1), 0)
            oh = (iota_c == idx_row).astype(jnp.float32)
            acc_ref[pl.ds(base, W), :] += jnp.dot(
                oh, fw_ref[...], preferred_element_type=jnp.float32)

    @pl.when(blk == nblk - 1)
    def _emit():
        cnt = jnp.maximum(cnt_ref[0], 1.0)
        sp_out_ref[0] = acc_ref[:NSP, :] / cnt + seg_m_ref[...]


def kernel(pts_feat, sp_idx, all_xyz, W1, b1, g1, be1, W2, b2, g2, be2,
           W3, b3, g3, be3, W4):
    Bb, Nn, C = pts_feat.shape
    nblk = Nn // K
    nb = Bb * nblk
    P = Bb * Nn

    idx32 = sp_idx.astype(jnp.int32)
    xyz = all_xyz.reshape(P, 3)
    offs = np.arange(Bb) * NSP
    total = Bb * NSP
    gidx = (idx32 + jnp.asarray(offs, jnp.int32)[:, None]).reshape(P)

    mesh = plsc.VectorSubcoreMesh(core_axis_name="c", subcore_axis_name="s",
                                  num_cores=2, num_subcores=16)
    f32 = jnp.float32
    sc_stats = pl.kernel(
        _sc_stats_body,
        out_type=[
            jax.ShapeDtypeStruct((NTILES, CHUNK * 4), f32),
            jax.ShapeDtypeStruct((Bb, NSP), f32),
        ],
        mesh=mesh,
        compiler_params=pltpu.CompilerParams(needs_layout_passes=False),
        scratch_types=[
            pltpu.VMEM((CHUNK * 4,), f32),
            pltpu.VMEM((CHUNK,), jnp.int32),
            pltpu.VMEM((NSP * 16,), f32),
            pltpu.VMEM((NSP * 16,), f32),
            pltpu.VMEM((NSP,), f32),
            pltpu.VMEM((NSP,), f32),
            pltpu.VMEM((NSP,), f32),
            pltpu.VMEM((NSP,), f32),
            pltpu.VMEM((NSP,), f32),
            pltpu.VMEM_SHARED((16, NSP * 16), f32),
        ],
    )
    xyz4 = jnp.pad(xyz, ((0, 0), (0, 1))).reshape(NTILES, CHUNK * 4)
    xyzn, cntb = sc_stats(xyz4, idx32.reshape(NTILES, CHUNK))
    xyz_p = xyzn.reshape(P, 4)
    cnt = cntb.reshape(total)

    grid = (nb,)
    full = lambda r, c: pl.BlockSpec((r, c), lambda i: (0, 0))
    proj = pl.pallas_call(
        _mlp1_body,
        grid=grid,
        in_specs=[
            pl.BlockSpec((K, 4), lambda i: (i, 0)),
            full(4, C), full(1, C), full(1, C), full(1, C),
            full(C, C), full(1, C), full(1, C), full(1, C),
        ],
        out_specs=pl.BlockSpec((K, 128), lambda i: (i, 0)),
        out_shape=jax.ShapeDtypeStruct((P, 128), jnp.float32),
    )(xyz_p, jnp.pad(W1, ((0, 1), (0, 0))), b1[None], g1[None], be1[None],
      W2, b2[None], g2[None], be2[None])

    sc_segmax = pl.kernel(
        _sc_segmax_body,
        out_type=jax.ShapeDtypeStruct((NTILES, NSP * 96), f32),
        mesh=mesh,
        compiler_params=pltpu.CompilerParams(needs_layout_passes=False),
        scratch_types=[
            pltpu.VMEM((SCCH * 128,), f32),
            pltpu.VMEM((CHUNK,), jnp.int32),
            pltpu.VMEM((NSP * 96,), f32),
            pltpu.VMEM((96,), f32),
        ],
    )
    slabs = sc_segmax(proj.reshape(NTILES, CHUNK * 128),
                      idx32.reshape(NTILES, CHUNK))
    slabs = slabs.reshape(Bb, QPB, NSP, C)

    idx_blk = idx32.reshape(nb, 1, K)
    s0 = idx_blk[:, :, 0][..., None]
    s1 = idx_blk[:, :, K - 1][..., None]
    cnt_in = cnt.reshape(Bb, NSP, 1)
    W3a, W3b = W3[:C], W3[C:]

    smem11 = pl.BlockSpec((1, 1, 1), lambda i: (i, 0, 0),
                          memory_space=pltpu.SMEM)
    w_flat, sp_feat_all = pl.pallas_call(
        functools.partial(_pass_c_body, nblk=nblk),
        grid=grid,
        in_specs=[
            smem11, smem11,
            pl.BlockSpec((1, 1, K), lambda i: (i, 0, 0)),
            pl.BlockSpec((K, 128), lambda i: (i, 0)),
            pl.BlockSpec((K, C), lambda i: (i, 0)),
            pl.BlockSpec((1, QPB, NSP, C), lambda i: (i // nblk, 0, 0, 0)),
            pl.BlockSpec((1, NSP, 1), lambda i: (i // nblk, 0, 0)),
            full(128, C), full(C, C), full(1, C), full(1, C), full(1, C),
            pl.BlockSpec((C, 1), lambda i: (0, 0)),
        ],
        out_specs=[
            pl.BlockSpec((K, 1), lambda i: (i, 0)),
            pl.BlockSpec((1, NSP, C), lambda i: (i // nblk, 0, 0)),
        ],
        out_shape=[
            jax.ShapeDtypeStruct((P, 1), jnp.float32),
            jax.ShapeDtypeStruct((Bb, NSP, C), jnp.float32),
        ],
        scratch_shapes=[
            pltpu.VMEM((TPAD, C), jnp.float32),
            pltpu.VMEM((TPAD, C), jnp.float32),
            pltpu.VMEM((K, C), jnp.float32),
            pltpu.VMEM((K, C), jnp.float32),
            pltpu.VMEM((NSP, C), jnp.float32),
        ],
    )(s0, s1, idx_blk, proj, pts_feat.reshape(P, C), slabs, cnt_in,
      jnp.pad(W3a, ((0, 32), (0, 0))), W3b, b3[None], g3[None], be3[None],
      W4)

    return (*tuple(sp_feat_all[i] for i in range(Bb)),
            w_flat.reshape(Bb, Nn, 1))


# speedup vs baseline: 2.7347x; 2.7347x over previous
import functools

import jax
import jax.numpy as jnp
import numpy as np
from jax import lax
from jax.experimental import pallas as pl
from jax.experimental.pallas import tpu as pltpu
from jax.experimental.pallas import tpu_sc as plsc

NSP = 1024
NTILES = 32
CHUNK = 12500
QPB = 4
K = 5000
W = 256
TPAD = NSP + W

_I16 = lambda: lax.broadcasted_iota(jnp.int32, (16,), 0)


def _stat_combine(a, b, min_m, max_m):
    return jnp.where(min_m, jnp.minimum(a, b),
                     jnp.where(max_m, jnp.maximum(a, b), a + b))


def _sc_stats_body(xyz_hbm, idx_hbm, xyzn_hbm, cnt_hbm, xyz_v, idx_v, tbl_v,
                   sib_v, ctrx_v, ctry_v, ctrz_v, inv_v, cnt_v, shared_v):
    c = lax.axis_index("c")
    s = lax.axis_index("s")
    wid = c * 16 + s
    b = wid // QPB
    iota16 = _I16()
    min_m = iota16 < 4
    max_m = (iota16 >= 4) & (iota16 < 8)
    one_m = iota16 == 11
    zero_m = iota16 >= 12
    init_row = jnp.where(min_m, jnp.inf, jnp.where(max_m, -jnp.inf, 0.0))
    colp = jnp.where((iota16 < 12) & (iota16 % 4 != 3), iota16 % 4, 0)

    pltpu.sync_copy(xyz_hbm.at[wid], xyz_v)
    pltpu.sync_copy(idx_hbm.at[wid], idx_v)

    def _init(i, carry):
        tbl_v[pl.ds(i * 16, 16)] = init_row
        return carry
    lax.fori_loop(0, NSP, _init, 0)

    def _pt_one(p, sg):
        g = plsc.load_gather(xyz_v, [4 * p + colp])
        pv = jnp.where(one_m, 1.0, jnp.where(zero_m, 0.0, g))
        r = pl.ds(sg * 16, 16)
        tbl_v[r] = _stat_combine(tbl_v[r], pv, min_m, max_m)

    def _pt(k, carry):
        p0 = k * 16
        iv = idx_v[pl.ds(p0, 16)]
        for j in range(16):
            _pt_one(p0 + j, iv[j])
        return carry
    lax.fori_loop(0, CHUNK // 16, _pt, 0)
    ivt = idx_v[pl.ds(CHUNK - 16, 16)]
    for j in range(16 - CHUNK % 16, 16):
        _pt_one(CHUNK - 16 + j, ivt[j])

    pltpu.sync_copy(tbl_v, shared_v.at[s])
    plsc.subcore_barrier()
    base = (s // QPB) * QPB
    pltpu.sync_copy(shared_v.at[base], tbl_v)
    for o in range(1, QPB):
        pltpu.sync_copy(shared_v.at[base + o], sib_v)

        def _mrg(i, carry):
            r = pl.ds(i * 16, 16)
            tbl_v[r] = _stat_combine(tbl_v[r], sib_v[r], min_m, max_m)
            return carry
        lax.fori_loop(0, NSP, _mrg, 0)

    def _nt(k, carry):
        s16 = k * 16 + iota16

        def col(cix):
            return plsc.load_gather(tbl_v, [s16 * 16 + cix])
        diam = jnp.maximum(jnp.maximum(col(4) - col(0), col(5) - col(1)),
                           col(6) - col(2))
        ct = col(11)
        cc = jnp.maximum(ct, 1.0)
        sl = pl.ds(k * 16, 16)
        ctrx_v[sl] = col(8) / cc
        ctry_v[sl] = col(9) / cc
        ctrz_v[sl] = col(10) / cc
        inv_v[sl] = 1.0 / (diam + 0.01)
        cnt_v[sl] = ct
        return carry
    lax.fori_loop(0, NSP // 16, _nt, 0)

    def _xn(k, carry):
        p16 = jnp.minimum(k * 16 + iota16, CHUNK - 1)
        i16 = plsc.load_gather(idx_v, [p16])
        cx = plsc.load_gather(ctrx_v, [i16])
        cy = plsc.load_gather(ctry_v, [i16])
        cz = plsc.load_gather(ctrz_v, [i16])
        iv = plsc.load_gather(inv_v, [i16])
        xs = plsc.load_gather(xyz_v, [4 * p16])
        ys = plsc.load_gather(xyz_v, [4 * p16 + 1])
        zs = plsc.load_gather(xyz_v, [4 * p16 + 2])
        plsc.store_scatter(xyz_v, [4 * p16], (xs - cx) * iv)
        plsc.store_scatter(xyz_v, [4 * p16 + 1], (ys - cy) * iv)
        plsc.store_scatter(xyz_v, [4 * p16 + 2], (zs - cz) * iv)
        return carry
    lax.fori_loop(0, (CHUNK + 15) // 16, _xn, 0)

    pltpu.sync_copy(xyz_v, xyzn_hbm.at[wid])

    @pl.when(wid % QPB == 0)
    def _():
        pltpu.sync_copy(cnt_v, cnt_hbm.at[b])


def _lnk(x, g, b, eps=1e-5):
    m = jnp.mean(x, axis=-1, keepdims=True)
    v = jnp.mean((x - m) ** 2, axis=-1, keepdims=True)
    return (x - m) / jnp.sqrt(v + eps) * g + b


def _mlp1_body(xyz_ref, w1_ref, b1_ref, g1_ref, be1_ref, w2_ref, b2_ref,
               g2_ref, be2_ref, out_ref):
    h = xyz_ref[...] @ w1_ref[...] + b1_ref[...]
    h = jnp.maximum(_lnk(h, g1_ref[...], be1_ref[...]), 0.0)
    p = h @ w2_ref[...] + b2_ref[...]
    out_ref[...] = _lnk(p, g2_ref[...], be2_ref[...])


def _pass_c_body(s0_ref, s1_ref, idx_ref, proj_ref, feat_ref, seg_ref,
                 cnt_ref, w3a_ref, w3b_ref, b3_ref, g3_ref, be3_ref, w4_ref,
                 w_out_ref, sp_out_ref, acc_ref, t_ref, g_ref, fw_ref, *,
                 nblk):
    i = pl.program_id(0)
    blk = i % nblk

    @pl.when(blk == 0)
    def _init():
        seg = seg_ref[0]
        seg_clean = jnp.where(jnp.isfinite(seg), seg, 0.0)
        t_ref[...] = seg_clean @ w3b_ref[...] + b3_ref[...]
        acc_ref[...] = jnp.zeros_like(acc_ref)

    s0 = s0_ref[0, 0, 0] & ~7
    s1 = s1_ref[0, 0, 0]

    for j in range(4):
        base = s0 + j * W

        @pl.when(base <= s1)
        def _gather():
            idx_col = idx_ref[0, 0, :][:, None]
            iota_w = base + lax.broadcasted_iota(jnp.int32, (1, W), 1)
            oh = (idx_col == iota_w).astype(jnp.float32)
            contrib = jnp.dot(oh, t_ref[pl.ds(base, W), :],
                              preferred_element_type=jnp.float32)
            if j == 0:
                g_ref[...] = contrib
            else:
                g_ref[...] += contrib

    y = proj_ref[...] @ w3a_ref[...] + g_ref[...]
    h2 = jnp.maximum(_lnk(y, g3_ref[...], be3_ref[...]), 0.0)
    w = jax.nn.sigmoid(h2 @ w4_ref[...]) * 2.0
    w_out_ref[...] = w
    fw_ref[...] = feat_ref[...] * w

    for j in range(4):
        base = s0 + j * W

        @pl.when(base <= s1)
        def _scatter():
            idx_row = idx_ref[0, 0, :][None, :]
            iota_c = base + lax.broadcasted_iota(jnp.int32, (W, 1), 0)
            oh = (iota_c == idx_row).astype(jnp.float32)
            acc_ref[pl.ds(base, W), :] += jnp.dot(
                oh, fw_ref[...], preferred_element_type=jnp.float32)

    @pl.when(blk == nblk - 1)
    def _emit():
        cnt = jnp.maximum(cnt_ref[0], 1.0)
        sp_out_ref[0] = acc_ref[:NSP, :] / cnt + seg_ref[0, :NSP, :]


def kernel(pts_feat, sp_idx, all_xyz, W1, b1, g1, be1, W2, b2, g2, be2,
           W3, b3, g3, be3, W4):
    Bb, Nn, C = pts_feat.shape
    nblk = Nn // K
    nb = Bb * nblk
    P = Bb * Nn

    idx32 = sp_idx.astype(jnp.int32)
    xyz = all_xyz.reshape(P, 3)
    offs = np.arange(Bb) * NSP
    total = Bb * NSP
    gidx = (idx32 + jnp.asarray(offs, jnp.int32)[:, None]).reshape(P)

    mesh = plsc.VectorSubcoreMesh(core_axis_name="c", subcore_axis_name="s",
                                  num_cores=2, num_subcores=16)
    f32 = jnp.float32
    sc_stats = pl.kernel(
        _sc_stats_body,
        out_type=[
            jax.ShapeDtypeStruct((NTILES, CHUNK * 4), f32),
            jax.ShapeDtypeStruct((Bb, NSP), f32),
        ],
        mesh=mesh,
        compiler_params=pltpu.CompilerParams(needs_layout_passes=False),
        scratch_types=[
            pltpu.VMEM((CHUNK * 4,), f32),
            pltpu.VMEM((CHUNK,), jnp.int32),
            pltpu.VMEM((NSP * 16,), f32),
            pltpu.VMEM((NSP * 16,), f32),
            pltpu.VMEM((NSP,), f32),
            pltpu.VMEM((NSP,), f32),
            pltpu.VMEM((NSP,), f32),
            pltpu.VMEM((NSP,), f32),
            pltpu.VMEM((NSP,), f32),
            pltpu.VMEM_SHARED((16, NSP * 16), f32),
        ],
    )
    xyz4 = jnp.pad(xyz, ((0, 0), (0, 1))).reshape(NTILES, CHUNK * 4)
    xyzn, cntb = sc_stats(xyz4, idx32.reshape(NTILES, CHUNK))
    xyz_p = xyzn.reshape(P, 4)
    cnt = cntb.reshape(total)

    grid = (nb,)
    full = lambda r, c: pl.BlockSpec((r, c), lambda i: (0, 0))
    proj = pl.pallas_call(
        _mlp1_body,
        grid=grid,
        in_specs=[
            pl.BlockSpec((K, 4), lambda i: (i, 0)),
            full(4, C), full(1, C), full(1, C), full(1, C),
            full(C, C), full(1, C), full(1, C), full(1, C),
        ],
        out_specs=pl.BlockSpec((K, C), lambda i: (i, 0)),
        out_shape=jax.ShapeDtypeStruct((P, C), jnp.float32),
    )(xyz_p, jnp.pad(W1, ((0, 1), (0, 0))), b1[None], g1[None], be1[None],
      W2, b2[None], g2[None], be2[None])

    seg = jax.ops.segment_max(proj, gidx, num_segments=total)

    seg_pad = jnp.pad(seg.reshape(Bb, NSP, C), ((0, 0), (0, W), (0, 0)))
    idx_blk = idx32.reshape(nb, 1, K)
    s0 = idx_blk[:, :, 0][..., None]
    s1 = idx_blk[:, :, K - 1][..., None]
    cnt_in = cnt.reshape(Bb, NSP, 1)
    W3a, W3b = W3[:C], W3[C:]

    smem11 = pl.BlockSpec((1, 1, 1), lambda i: (i, 0, 0),
                          memory_space=pltpu.SMEM)
    w_flat, sp_feat_all = pl.pallas_call(
        functools.partial(_pass_c_body, nblk=nblk),
        grid=grid,
        in_specs=[
            smem11, smem11,
            pl.BlockSpec((1, 1, K), lambda i: (i, 0, 0)),
            pl.BlockSpec((K, C), lambda i: (i, 0)),
            pl.BlockSpec((K, C), lambda i: (i, 0)),
            pl.BlockSpec((1, TPAD, C), lambda i: (i // nblk, 0, 0)),
            pl.BlockSpec((1, NSP, 1), lambda i: (i // nblk, 0, 0)),
            full(C, C), full(C, C), full(1, C), full(1, C), full(1, C),
            pl.BlockSpec((C, 1), lambda i: (0, 0)),
        ],
        out_specs=[
            pl.BlockSpec((K, 1), lambda i: (i, 0)),
            pl.BlockSpec((1, NSP, C), lambda i: (i // nblk, 0, 0)),
        ],
        out_shape=[
            jax.ShapeDtypeStruct((P, 1), jnp.float32),
            jax.ShapeDtypeStruct((Bb, NSP, C), jnp.float32),
        ],
        scratch_shapes=[
            pltpu.VMEM((TPAD, C), jnp.float32),
            pltpu.VMEM((TPAD, C), jnp.float32),
            pltpu.VMEM((K, C), jnp.float32),
            pltpu.VMEM((K, C), jnp.float32),
        ],
    )(s0, s1, idx_blk, proj, pts_feat.reshape(P, C), seg_pad, cnt_in,
      W3a, W3b, b3[None], g3[None], be3[None], W4)

    return (*tuple(sp_feat_all[i] for i in range(Bb)),
            w_flat.reshape(Bb, Nn, 1))


# speedup vs baseline: 2.8026x; 1.0248x over previous
import functools

import jax
import jax.numpy as jnp
import numpy as np
from jax import lax
from jax.experimental import pallas as pl
from jax.experimental.pallas import tpu as pltpu
from jax.experimental.pallas import tpu_sc as plsc

NSP = 1024
NTILES = 32
CHUNK = 12500
QPB = 4
K = 5000
W = 256
TPAD = NSP + W

_I16 = lambda: lax.broadcasted_iota(jnp.int32, (16,), 0)


def _stat_combine(a, b, min_m, max_m):
    return jnp.where(min_m, jnp.minimum(a, b),
                     jnp.where(max_m, jnp.maximum(a, b), a + b))


def _sc_stats_body(xyz_hbm, idx_hbm, xyzn_hbm, cnt_hbm, xyz_v, idx_v, tbl_v,
                   sib_v, ctrx_v, ctry_v, ctrz_v, inv_v, cnt_v, shared_v):
    c = lax.axis_index("c")
    s = lax.axis_index("s")
    wid = c * 16 + s
    b = wid // QPB
    iota16 = _I16()
    min_m = iota16 < 4
    max_m = (iota16 >= 4) & (iota16 < 8)
    one_m = iota16 == 11
    zero_m = iota16 >= 12
    init_row = jnp.where(min_m, jnp.inf, jnp.where(max_m, -jnp.inf, 0.0))
    colp = jnp.where((iota16 < 12) & (iota16 % 4 != 3), iota16 % 4, 0)

    pltpu.sync_copy(xyz_hbm.at[wid], xyz_v)
    pltpu.sync_copy(idx_hbm.at[wid], idx_v)

    def _init(i, carry):
        tbl_v[pl.ds(i * 16, 16)] = init_row
        return carry
    lax.fori_loop(0, NSP, _init, 0)

    def _pt_one(p, sg):
        g = plsc.load_gather(xyz_v, [4 * p + colp])
        pv = jnp.where(one_m, 1.0, jnp.where(zero_m, 0.0, g))
        r = pl.ds(sg * 16, 16)
        tbl_v[r] = _stat_combine(tbl_v[r], pv, min_m, max_m)

    def _pt(k, carry):
        p0 = k * 16
        iv = idx_v[pl.ds(p0, 16)]
        for j in range(16):
            _pt_one(p0 + j, iv[j])
        return carry
    lax.fori_loop(0, CHUNK // 16, _pt, 0)
    ivt = idx_v[pl.ds(CHUNK - 16, 16)]
    for j in range(16 - CHUNK % 16, 16):
        _pt_one(CHUNK - 16 + j, ivt[j])

    pltpu.sync_copy(tbl_v, shared_v.at[s])
    plsc.subcore_barrier()
    base = (s // QPB) * QPB
    pltpu.sync_copy(shared_v.at[base], tbl_v)
    for o in range(1, QPB):
        pltpu.sync_copy(shared_v.at[base + o], sib_v)

        def _mrg(i, carry):
            r = pl.ds(i * 16, 16)
            tbl_v[r] = _stat_combine(tbl_v[r], sib_v[r], min_m, max_m)
            return carry
        lax.fori_loop(0, NSP, _mrg, 0)

    def _nt(k, carry):
        s16 = k * 16 + iota16

        def col(cix):
            return plsc.load_gather(tbl_v, [s16 * 16 + cix])
        diam = jnp.maximum(jnp.maximum(col(4) - col(0), col(5) - col(1)),
                           col(6) - col(2))
        ct = col(11)
        cc = jnp.maximum(ct, 1.0)
        sl = pl.ds(k * 16, 16)
        ctrx_v[sl] = col(8) / cc
        ctry_v[sl] = col(9) / cc
        ctrz_v[sl] = col(10) / cc
        inv_v[sl] = 1.0 / (diam + 0.01)
        cnt_v[sl] = ct
        return carry
    lax.fori_loop(0, NSP // 16, _nt, 0)

    def _xn(k, carry):
        p16 = jnp.minimum(k * 16 + iota16, CHUNK - 1)
        i16 = plsc.load_gather(idx_v, [p16])
        cx = plsc.load_gather(ctrx_v, [i16])
        cy = plsc.load_gather(ctry_v, [i16])
        cz = plsc.load_gather(ctrz_v, [i16])
        iv = plsc.load_gather(inv_v, [i16])
        xs = plsc.load_gather(xyz_v, [4 * p16])
        ys = plsc.load_gather(xyz_v, [4 * p16 + 1])
        zs = plsc.load_gather(xyz_v, [4 * p16 + 2])
        plsc.store_scatter(xyz_v, [4 * p16], (xs - cx) * iv)
        plsc.store_scatter(xyz_v, [4 * p16 + 1], (ys - cy) * iv)
        plsc.store_scatter(xyz_v, [4 * p16 + 2], (zs - cz) * iv)
        return carry
    lax.fori_loop(0, (CHUNK + 15) // 16, _xn, 0)

    pltpu.sync_copy(xyz_v, xyzn_hbm.at[wid])

    @pl.when(wid % QPB == 0)
    def _():
        pltpu.sync_copy(cnt_v, cnt_hbm.at[b])


def _lnk(x, g, b, eps=1e-5):
    m = jnp.mean(x, axis=-1, keepdims=True)
    v = jnp.mean((x - m) ** 2, axis=-1, keepdims=True)
    return (x - m) / jnp.sqrt(v + eps) * g + b


def _mlp1_body(xyz_ref, w1_ref, b1_ref, g1_ref, be1_ref, w2_ref, b2_ref,
               g2_ref, be2_ref, out_ref):
    h = xyz_ref[...] @ w1_ref[...] + b1_ref[...]
    h = jnp.maximum(_lnk(h, g1_ref[...], be1_ref[...]), 0.0)
    p = jnp.dot(h.astype(jnp.bfloat16), w2_ref[...].astype(jnp.bfloat16),
                preferred_element_type=jnp.float32) + b2_ref[...]
    out_ref[...] = _lnk(p, g2_ref[...], be2_ref[...])


def _pass_c_body(s0_ref, s1_ref, idx_ref, proj_ref, feat_ref, seg_ref,
                 cnt_ref, w3a_ref, w3b_ref, b3_ref, g3_ref, be3_ref, w4_ref,
                 w_out_ref, sp_out_ref, acc_ref, t_ref, g_ref, fw_ref, *,
                 nblk):
    i = pl.program_id(0)
    blk = i % nblk

    @pl.when(blk == 0)
    def _init():
        seg = seg_ref[0]
        seg_clean = jnp.where(jnp.isfinite(seg), seg, 0.0)
        t_ref[...] = seg_clean @ w3b_ref[...] + b3_ref[...]
        acc_ref[...] = jnp.zeros_like(acc_ref)

    s0 = s0_ref[0, 0, 0] & ~7
    s1 = s1_ref[0, 0, 0]

    for j in range(4):
        base = s0 + j * W

        @pl.when(base <= s1)
        def _gather():
            idx_col = idx_ref[0, 0, :][:, None]
            iota_w = base + lax.broadcasted_iota(jnp.int32, (1, W), 1)
            oh = (idx_col == iota_w).astype(jnp.bfloat16)
            contrib = jnp.dot(oh,
                              t_ref[pl.ds(base, W), :].astype(jnp.bfloat16),
                              preferred_element_type=jnp.float32)
            if j == 0:
                g_ref[...] = contrib
            else:
                g_ref[...] += contrib

    y = jnp.dot(proj_ref[...].astype(jnp.bfloat16),
                w3a_ref[...].astype(jnp.bfloat16),
                preferred_element_type=jnp.float32) + g_ref[...]
    h2 = jnp.maximum(_lnk(y, g3_ref[...], be3_ref[...]), 0.0)
    w = jax.nn.sigmoid(h2 @ w4_ref[...]) * 2.0
    w_out_ref[...] = w
    fw_ref[...] = feat_ref[...] * w

    for j in range(4):
        base = s0 + j * W

        @pl.when(base <= s1)
        def _scatter():
            idx_row = idx_ref[0, 0, :][None, :]
            iota_c = base + lax.broadcasted_iota(jnp.int32, (W, 1), 0)
            oh = (iota_c == idx_row).astype(jnp.bfloat16)
            acc_ref[pl.ds(base, W), :] += jnp.dot(
                oh, fw_ref[...].astype(jnp.bfloat16),
                preferred_element_type=jnp.float32)

    @pl.when(blk == nblk - 1)
    def _emit():
        cnt = jnp.maximum(cnt_ref[0], 1.0)
        sp_out_ref[0] = acc_ref[:NSP, :] / cnt + seg_ref[0, :NSP, :]


def kernel(pts_feat, sp_idx, all_xyz, W1, b1, g1, be1, W2, b2, g2, be2,
           W3, b3, g3, be3, W4):
    Bb, Nn, C = pts_feat.shape
    nblk = Nn // K
    nb = Bb * nblk
    P = Bb * Nn

    idx32 = sp_idx.astype(jnp.int32)
    xyz = all_xyz.reshape(P, 3)
    offs = np.arange(Bb) * NSP
    total = Bb * NSP
    gidx = (idx32 + jnp.asarray(offs, jnp.int32)[:, None]).reshape(P)

    mesh = plsc.VectorSubcoreMesh(core_axis_name="c", subcore_axis_name="s",
                                  num_cores=2, num_subcores=16)
    f32 = jnp.float32
    sc_stats = pl.kernel(
        _sc_stats_body,
        out_type=[
            jax.ShapeDtypeStruct((NTILES, CHUNK * 4), f32),
            jax.ShapeDtypeStruct((Bb, NSP), f32),
        ],
        mesh=mesh,
        compiler_params=pltpu.CompilerParams(needs_layout_passes=False),
        scratch_types=[
            pltpu.VMEM((CHUNK * 4,), f32),
            pltpu.VMEM((CHUNK,), jnp.int32),
            pltpu.VMEM((NSP * 16,), f32),
            pltpu.VMEM((NSP * 16,), f32),
            pltpu.VMEM((NSP,), f32),
            pltpu.VMEM((NSP,), f32),
            pltpu.VMEM((NSP,), f32),
            pltpu.VMEM((NSP,), f32),
            pltpu.VMEM((NSP,), f32),
            pltpu.VMEM_SHARED((16, NSP * 16), f32),
        ],
    )
    xyz4 = jnp.pad(xyz, ((0, 0), (0, 1))).reshape(NTILES, CHUNK * 4)
    xyzn, cntb = sc_stats(xyz4, idx32.reshape(NTILES, CHUNK))
    xyz_p = xyzn.reshape(P, 4)
    cnt = cntb.reshape(total)

    grid = (nb,)
    full = lambda r, c: pl.BlockSpec((r, c), lambda i: (0, 0))
    proj = pl.pallas_call(
        _mlp1_body,
        grid=grid,
        in_specs=[
            pl.BlockSpec((K, 4), lambda i: (i, 0)),
            full(4, C), full(1, C), full(1, C), full(1, C),
            full(C, C), full(1, C), full(1, C), full(1, C),
        ],
        out_specs=pl.BlockSpec((K, C), lambda i: (i, 0)),
        out_shape=jax.ShapeDtypeStruct((P, C), jnp.float32),
    )(xyz_p, jnp.pad(W1, ((0, 1), (0, 0))), b1[None], g1[None], be1[None],
      W2, b2[None], g2[None], be2[None])

    seg = jax.ops.segment_max(proj.astype(jnp.bfloat16), gidx,
                              num_segments=total).astype(jnp.float32)

    seg_pad = jnp.pad(seg.reshape(Bb, NSP, C), ((0, 0), (0, W), (0, 0)))
    idx_blk = idx32.reshape(nb, 1, K)
    s0 = idx_blk[:, :, 0][..., None]
    s1 = idx_blk[:, :, K - 1][..., None]
    cnt_in = cnt.reshape(Bb, NSP, 1)
    W3a, W3b = W3[:C], W3[C:]

    smem11 = pl.BlockSpec((1, 1, 1), lambda i: (i, 0, 0),
                          memory_space=pltpu.SMEM)
    w_flat, sp_feat_all = pl.pallas_call(
        functools.partial(_pass_c_body, nblk=nblk),
        grid=grid,
        in_specs=[
            smem11, smem11,
            pl.BlockSpec((1, 1, K), lambda i: (i, 0, 0)),
            pl.BlockSpec((K, C), lambda i: (i, 0)),
            pl.BlockSpec((K, C), lambda i: (i, 0)),
            pl.BlockSpec((1, TPAD, C), lambda i: (i // nblk, 0, 0)),
            pl.BlockSpec((1, NSP, 1), lambda i: (i // nblk, 0, 0)),
            full(C, C), full(C, C), full(1, C), full(1, C), full(1, C),
            pl.BlockSpec((C, 1), lambda i: (0, 0)),
        ],
        out_specs=[
            pl.BlockSpec((K, 1), lambda i: (i, 0)),
            pl.BlockSpec((1, NSP, C), lambda i: (i // nblk, 0, 0)),
        ],
        out_shape=[
            jax.ShapeDtypeStruct((P, 1), jnp.float32),
            jax.ShapeDtypeStruct((Bb, NSP, C), jnp.float32),
        ],
        scratch_shapes=[
            pltpu.VMEM((TPAD, C), jnp.float32),
            pltpu.VMEM((TPAD, C), jnp.float32),
            pltpu.VMEM((K, C), jnp.float32),
            pltpu.VMEM((K, C), jnp.float32),
        ],
    )(s0, s1, idx_blk, proj, pts_feat.reshape(P, C), seg_pad, cnt_in,
      W3a, W3b, b3[None], g3[None], be3[None], W4)

    return (*tuple(sp_feat_all[i] for i in range(Bb)),
            w_flat.reshape(Bb, Nn, 1))


# speedup vs baseline: 2.9417x; 1.0496x over previous
import functools

import jax
import jax.numpy as jnp
import numpy as np
from jax import lax
from jax.experimental import pallas as pl
from jax.experimental.pallas import tpu as pltpu
from jax.experimental.pallas import tpu_sc as plsc

NSP = 1024
NTILES = 32
CHUNK = 12500
QPB = 4
K = 10000
W = 256
TPAD = NSP + W

_I16 = lambda: lax.broadcasted_iota(jnp.int32, (16,), 0)


def _stat_combine(a, b, min_m, max_m):
    return jnp.where(min_m, jnp.minimum(a, b),
                     jnp.where(max_m, jnp.maximum(a, b), a + b))


def _sc_stats_body(xyz_hbm, idx_hbm, xyzn_hbm, cnt_hbm, xyz_v, idx_v, tbl_v,
                   sib_v, ctrx_v, ctry_v, ctrz_v, inv_v, cnt_v, shared_v):
    c = lax.axis_index("c")
    s = lax.axis_index("s")
    wid = c * 16 + s
    b = wid // QPB
    iota16 = _I16()
    min_m = iota16 < 4
    max_m = (iota16 >= 4) & (iota16 < 8)
    one_m = iota16 == 11
    zero_m = iota16 >= 12
    init_row = jnp.where(min_m, jnp.inf, jnp.where(max_m, -jnp.inf, 0.0))
    colp = jnp.where((iota16 < 12) & (iota16 % 4 != 3), iota16 % 4, 0)

    pltpu.sync_copy(xyz_hbm.at[wid], xyz_v)
    pltpu.sync_copy(idx_hbm.at[wid], idx_v)

    def _init(i, carry):
        tbl_v[pl.ds(i * 16, 16)] = init_row
        return carry
    lax.fori_loop(0, NSP, _init, 0)

    def _pt_one(p, sg):
        g = plsc.load_gather(xyz_v, [4 * p + colp])
        pv = jnp.where(one_m, 1.0, jnp.where(zero_m, 0.0, g))
        r = pl.ds(sg * 16, 16)
        tbl_v[r] = _stat_combine(tbl_v[r], pv, min_m, max_m)

    def _pt(k, carry):
        p0 = k * 16
        iv = idx_v[pl.ds(p0, 16)]
        for j in range(16):
            _pt_one(p0 + j, iv[j])
        return carry
    lax.fori_loop(0, CHUNK // 16, _pt, 0)
    ivt = idx_v[pl.ds(CHUNK - 16, 16)]
    for j in range(16 - CHUNK % 16, 16):
        _pt_one(CHUNK - 16 + j, ivt[j])

    pltpu.sync_copy(tbl_v, shared_v.at[s])
    plsc.subcore_barrier()
    base = (s // QPB) * QPB
    pltpu.sync_copy(shared_v.at[base], tbl_v)
    for o in range(1, QPB):
        pltpu.sync_copy(shared_v.at[base + o], sib_v)

        def _mrg(i, carry):
            r = pl.ds(i * 16, 16)
            tbl_v[r] = _stat_combine(tbl_v[r], sib_v[r], min_m, max_m)
            return carry
        lax.fori_loop(0, NSP, _mrg, 0)

    def _nt(k, carry):
        s16 = k * 16 + iota16

        def col(cix):
            return plsc.load_gather(tbl_v, [s16 * 16 + cix])
        diam = jnp.maximum(jnp.maximum(col(4) - col(0), col(5) - col(1)),
                           col(6) - col(2))
        ct = col(11)
        cc = jnp.maximum(ct, 1.0)
        sl = pl.ds(k * 16, 16)
        ctrx_v[sl] = col(8) / cc
        ctry_v[sl] = col(9) / cc
        ctrz_v[sl] = col(10) / cc
        inv_v[sl] = 1.0 / (diam + 0.01)
        cnt_v[sl] = ct
        return carry
    lax.fori_loop(0, NSP // 16, _nt, 0)

    def _xn(k, carry):
        p16 = jnp.minimum(k * 16 + iota16, CHUNK - 1)
        i16 = plsc.load_gather(idx_v, [p16])
        cx = plsc.load_gather(ctrx_v, [i16])
        cy = plsc.load_gather(ctry_v, [i16])
        cz = plsc.load_gather(ctrz_v, [i16])
        iv = plsc.load_gather(inv_v, [i16])
        xs = plsc.load_gather(xyz_v, [4 * p16])
        ys = plsc.load_gather(xyz_v, [4 * p16 + 1])
        zs = plsc.load_gather(xyz_v, [4 * p16 + 2])
        plsc.store_scatter(xyz_v, [4 * p16], (xs - cx) * iv)
        plsc.store_scatter(xyz_v, [4 * p16 + 1], (ys - cy) * iv)
        plsc.store_scatter(xyz_v, [4 * p16 + 2], (zs - cz) * iv)
        return carry
    lax.fori_loop(0, (CHUNK + 15) // 16, _xn, 0)

    pltpu.sync_copy(xyz_v, xyzn_hbm.at[wid])

    @pl.when(wid % QPB == 0)
    def _():
        pltpu.sync_copy(cnt_v, cnt_hbm.at[b])


def _lnk(x, g, b, eps=1e-5):
    m = jnp.mean(x, axis=-1, keepdims=True)
    v = jnp.mean((x - m) ** 2, axis=-1, keepdims=True)
    return (x - m) / jnp.sqrt(v + eps) * g + b


def _mlp1_body(xyz_ref, w1_ref, b1_ref, g1_ref, be1_ref, w2_ref, b2_ref,
               g2_ref, be2_ref, out_ref):
    h = xyz_ref[...] @ w1_ref[...] + b1_ref[...]
    h = jnp.maximum(_lnk(h, g1_ref[...], be1_ref[...]), 0.0)
    p = jnp.dot(h.astype(jnp.bfloat16), w2_ref[...].astype(jnp.bfloat16),
                preferred_element_type=jnp.float32) + b2_ref[...]
    out_ref[...] = _lnk(p, g2_ref[...], be2_ref[...])


def _pass_c_body(s0_ref, s1_ref, idx_ref, proj_ref, feat_ref, seg_ref,
                 cnt_ref, w3a_ref, w3b_ref, b3_ref, g3_ref, be3_ref, w4_ref,
                 w_out_ref, sp_out_ref, acc_ref, t_ref, g_ref, fw_ref, *,
                 nblk):
    i = pl.program_id(0)
    blk = i % nblk

    @pl.when(blk == 0)
    def _init():
        seg = seg_ref[0]
        seg_clean = jnp.where(jnp.isfinite(seg), seg, 0.0)
        t_ref[...] = seg_clean @ w3b_ref[...] + b3_ref[...]
        acc_ref[...] = jnp.zeros_like(acc_ref)

    s0 = s0_ref[0, 0, 0] & ~7
    s1 = s1_ref[0, 0, 0]

    for j in range(4):
        base = s0 + j * W

        @pl.when(base <= s1)
        def _gather():
            idx_col = idx_ref[0, 0, :][:, None]
            iota_w = base + lax.broadcasted_iota(jnp.int32, (1, W), 1)
            oh = (idx_col == iota_w).astype(jnp.bfloat16)
            contrib = jnp.dot(oh,
                              t_ref[pl.ds(base, W), :].astype(jnp.bfloat16),
                              preferred_element_type=jnp.float32)
            if j == 0:
                g_ref[...] = contrib
            else:
                g_ref[...] += contrib

    y = jnp.dot(proj_ref[...].astype(jnp.bfloat16),
                w3a_ref[...].astype(jnp.bfloat16),
                preferred_element_type=jnp.float32) + g_ref[...]
    h2 = jnp.maximum(_lnk(y, g3_ref[...], be3_ref[...]), 0.0)
    w = jax.nn.sigmoid(h2 @ w4_ref[...]) * 2.0
    w_out_ref[...] = w
    fw_ref[...] = feat_ref[...] * w

    for j in range(4):
        base = s0 + j * W

        @pl.when(base <= s1)
        def _scatter():
            idx_row = idx_ref[0, 0, :][None, :]
            iota_c = base + lax.broadcasted_iota(jnp.int32, (W, 1), 0)
            oh = (iota_c == idx_row).astype(jnp.bfloat16)
            acc_ref[pl.ds(base, W), :] += jnp.dot(
                oh, fw_ref[...].astype(jnp.bfloat16),
                preferred_element_type=jnp.float32)

    @pl.when(blk == nblk - 1)
    def _emit():
        cnt = jnp.maximum(cnt_ref[0], 1.0)
        sp_out_ref[0] = acc_ref[:NSP, :] / cnt + seg_ref[0, :NSP, :]


def kernel(pts_feat, sp_idx, all_xyz, W1, b1, g1, be1, W2, b2, g2, be2,
           W3, b3, g3, be3, W4):
    Bb, Nn, C = pts_feat.shape
    nblk = Nn // K
    nb = Bb * nblk
    P = Bb * Nn

    idx32 = sp_idx.astype(jnp.int32)
    xyz = all_xyz.reshape(P, 3)
    offs = np.arange(Bb) * NSP
    total = Bb * NSP
    gidx = (idx32 + jnp.asarray(offs, jnp.int32)[:, None]).reshape(P)

    mesh = plsc.VectorSubcoreMesh(core_axis_name="c", subcore_axis_name="s",
                                  num_cores=2, num_subcores=16)
    f32 = jnp.float32
    sc_stats = pl.kernel(
        _sc_stats_body,
        out_type=[
            jax.ShapeDtypeStruct((NTILES, CHUNK * 4), f32),
            jax.ShapeDtypeStruct((Bb, NSP), f32),
        ],
        mesh=mesh,
        compiler_params=pltpu.CompilerParams(needs_layout_passes=False),
        scratch_types=[
            pltpu.VMEM((CHUNK * 4,), f32),
            pltpu.VMEM((CHUNK,), jnp.int32),
            pltpu.VMEM((NSP * 16,), f32),
            pltpu.VMEM((NSP * 16,), f32),
            pltpu.VMEM((NSP,), f32),
            pltpu.VMEM((NSP,), f32),
            pltpu.VMEM((NSP,), f32),
            pltpu.VMEM((NSP,), f32),
            pltpu.VMEM((NSP,), f32),
            pltpu.VMEM_SHARED((16, NSP * 16), f32),
        ],
    )
    xyz4 = jnp.pad(xyz, ((0, 0), (0, 1))).reshape(NTILES, CHUNK * 4)
    xyzn, cntb = sc_stats(xyz4, idx32.reshape(NTILES, CHUNK))
    xyz_p = xyzn.reshape(P, 4)
    cnt = cntb.reshape(total)

    grid = (nb,)
    full = lambda r, c: pl.BlockSpec((r, c), lambda i: (0, 0))
    proj = pl.pallas_call(
        _mlp1_body,
        grid=grid,
        in_specs=[
            pl.BlockSpec((K, 4), lambda i: (i, 0)),
            full(4, C), full(1, C), full(1, C), full(1, C),
            full(C, C), full(1, C), full(1, C), full(1, C),
        ],
        out_specs=pl.BlockSpec((K, C), lambda i: (i, 0)),
        out_shape=jax.ShapeDtypeStruct((P, C), jnp.float32),
    )(xyz_p, jnp.pad(W1, ((0, 1), (0, 0))), b1[None], g1[None], be1[None],
      W2, b2[None], g2[None], be2[None])

    seg = jax.ops.segment_max(proj.astype(jnp.bfloat16), gidx,
                              num_segments=total).astype(jnp.float32)

    seg_pad = jnp.pad(seg.reshape(Bb, NSP, C), ((0, 0), (0, W), (0, 0)))
    idx_blk = idx32.reshape(nb, 1, K)
    s0 = idx_blk[:, :, 0][..., None]
    s1 = idx_blk[:, :, K - 1][..., None]
    cnt_in = cnt.reshape(Bb, NSP, 1)
    W3a, W3b = W3[:C], W3[C:]

    smem11 = pl.BlockSpec((1, 1, 1), lambda i: (i, 0, 0),
                          memory_space=pltpu.SMEM)
    w_flat, sp_feat_all = pl.pallas_call(
        functools.partial(_pass_c_body, nblk=nblk),
        grid=grid,
        in_specs=[
            smem11, smem11,
            pl.BlockSpec((1, 1, K), lambda i: (i, 0, 0)),
            pl.BlockSpec((K, C), lambda i: (i, 0)),
            pl.BlockSpec((K, C), lambda i: (i, 0)),
            pl.BlockSpec((1, TPAD, C), lambda i: (i // nblk, 0, 0)),
            pl.BlockSpec((1, NSP, 1), lambda i: (i // nblk, 0, 0)),
            full(C, C), full(C, C), full(1, C), full(1, C), full(1, C),
            pl.BlockSpec((C, 1), lambda i: (0, 0)),
        ],
        out_specs=[
            pl.BlockSpec((K, 1), lambda i: (i, 0)),
            pl.BlockSpec((1, NSP, C), lambda i: (i // nblk, 0, 0)),
        ],
        out_shape=[
            jax.ShapeDtypeStruct((P, 1), jnp.float32),
            jax.ShapeDtypeStruct((Bb, NSP, C), jnp.float32),
        ],
        scratch_shapes=[
            pltpu.VMEM((TPAD, C), jnp.float32),
            pltpu.VMEM((TPAD, C), jnp.float32),
            pltpu.VMEM((K, C), jnp.float32),
            pltpu.VMEM((K, C), jnp.float32),
        ],
    )(s0, s1, idx_blk, proj, pts_feat.reshape(P, C), seg_pad, cnt_in,
      W3a, W3b, b3[None], g3[None], be3[None], W4)

    return (*tuple(sp_feat_all[i] for i in range(Bb)),
            w_flat.reshape(Bb, Nn, 1))
